# baseline (speedup 1.0000x reference)
import functools

import jax
import jax.numpy as jnp
import numpy as np
from jax import lax
from jax.experimental import pallas as pl
from jax.experimental.pallas import tpu as pltpu
from jax.experimental.pallas import tpu_sc as plsc
from jax._src import config as _jcfg

N_USERS = 15000
N_ITEMS = 10000
N_NODES = N_USERS + N_ITEMS
D = 128
LAYERS = 2

NEG_INF = np.float32(-1e30)
EPS = np.float32(1e-12)


def _flash_body(nk_valid, q_ref, k_ref, o_ref, acc_ref, m_ref, l_ref):
    j = pl.program_id(1)
    nj = pl.num_programs(1)

    @pl.when(j == 0)
    def _init():
        acc_ref[...] = jnp.zeros_like(acc_ref)
        m_ref[...] = jnp.full_like(m_ref, NEG_INF)
        l_ref[...] = jnp.zeros_like(l_ref)

    q = q_ref[...]
    qn = q * lax.reciprocal(jnp.sqrt(jnp.sum(q * q, axis=1, keepdims=True)) + EPS)
    k = k_ref[...]
    kn = k * lax.reciprocal(jnp.sqrt(jnp.sum(k * k, axis=1, keepdims=True)) + EPS)

    s = jax.lax.dot_general(qn.astype(jnp.bfloat16), kn.astype(jnp.bfloat16),
                            (((1,), (1,)), ((), ())),
                            preferred_element_type=jnp.float32)
    bk = k.shape[0]
    kidx = j * bk + lax.broadcasted_iota(jnp.int32, s.shape, 1)
    s = jnp.where(kidx < nk_valid, s, NEG_INF)

    m_prev = m_ref[...]
    l_prev = l_ref[...]
    m_cur = jnp.max(s, axis=1, keepdims=True)
    m_new = jnp.maximum(m_prev, m_cur)
    p = jnp.exp(s - m_new)
    corr = jnp.exp(m_prev - m_new)
    l_new = l_prev * corr + jnp.sum(p, axis=1, keepdims=True)
    m_ref[...] = m_new
    l_ref[...] = l_new
    pv = jax.lax.dot_general(p.astype(jnp.bfloat16), k.astype(jnp.bfloat16),
                             (((1,), (0,)), ((), ())),
                             preferred_element_type=jnp.float32)
    acc_ref[...] = acc_ref[...] * corr + pv

    @pl.when(j == nj - 1)
    def _fin():
        o_ref[...] = q + acc_ref[...] * lax.reciprocal(l_ref[...])


def _similarity_transfer(target, auxi, bq=512, bk=512):
    n, d = target.shape
    nk = auxi.shape[0]
    n_pad = (n + bq - 1) // bq * bq
    nk_pad = (nk + bk - 1) // bk * bk
    tq = jnp.pad(target, ((0, n_pad - n), (0, 0)))
    ta = jnp.pad(auxi, ((0, nk_pad - nk), (0, 0)))
    grid = (n_pad // bq, nk_pad // bk)
    out = pl.pallas_call(
        functools.partial(_flash_body, nk),
        grid=grid,
        in_specs=[
            pl.BlockSpec((bq, d), lambda i, j: (i, np.int32(0))),
            pl.BlockSpec((bk, d), lambda i, j: (j, np.int32(0))),
        ],
        out_specs=pl.BlockSpec((bq, d), lambda i, j: (i, np.int32(0))),
        out_shape=jax.ShapeDtypeStruct((n_pad, d), jnp.float32),
        scratch_shapes=[
            pltpu.VMEM((bq, d), jnp.float32),
            pltpu.VMEM((bq, 1), jnp.float32),
            pltpu.VMEM((bq, 1), jnp.float32),
        ],
        compiler_params=pltpu.CompilerParams(
            dimension_semantics=("arbitrary", "arbitrary"),
        ),
    )(tq, ta)
    return out[:n]


SC_NC = 2
SC_NS = 16
NP = 25088
H = D // 2
EP = 409600
ET = EP // SC_NS
EC = 128
NCH = ET // EC
RPT = NP // SC_NS
RC = 56
NRC = RPT // RC


def _prop_body(ego, srcr, dstr, wr, final, cur1,
               acc, rows0, rows1, src0, src1, src2, src3,
               w0, w1, w2, w3, dst0, dst1, cb1, cb2,
               isem0, isem1, isem2, isem3, dsem0, dsem1,
               gsem0, gsem1, ssem0, ssem1):
    c = lax.axis_index("c")
    s = lax.axis_index("s")
    r0 = s * np.int32(RPT)
    ebase = s * np.int32(ET)
    coff = c * np.int32(NP)

    def zero_cb1():
        def zb(e, carry):
            for kk in range(H // 16):
                cb1[e, pl.ds(kk * 16, 16)] = jnp.zeros((16,), jnp.float32)
            return carry
        lax.fori_loop(np.int32(0), np.int32(RC), zb, np.int32(0))

    def zero_acc():
        for j in range(NRC):
            pltpu.sync_copy(cb1, acc.at[pl.ds(r0 + np.int32(j * RC), RC)])

    def edge_pass(tab):
        srcs = (src0, src1, src2, src3)
        wbufs = (w0, w1, w2, w3)
        dsts = (dst0, dst1)
        rows = (rows0, rows1)
        isems = (isem0, isem1, isem2, isem3)
        dsems = (dsem0, dsem1)
        gsems = (gsem0, gsem1)
        ssems = (ssem0, ssem1)
        NCH4 = NCH // 4

        def fire_srcw(koff, j):
            pltpu.async_copy(srcr.at[pl.ds(koff, EC)], srcs[j], isems[j])
            pltpu.async_copy(wr.at[pl.ds(koff, EC)], wbufs[j], isems[j])

        def wait_srcw(j):
            pltpu.make_async_copy(srcr.at[pl.ds(ebase, EC)], srcs[j],
                                  isems[j]).wait()
            pltpu.make_async_copy(wr.at[pl.ds(ebase, EC)], wbufs[j],
                                  isems[j]).wait()

        def fire_dst(koff, b):
            pltpu.async_copy(dstr.at[pl.ds(koff, EC)], dsts[b], dsems[b])

        def wait_dst(b):
            pltpu.make_async_copy(dstr.at[pl.ds(ebase, EC)], dsts[b],
                                  dsems[b]).wait()

        def adjust(j):
            for g in range(EC // 16):
                srcs[j][pl.ds(g * 16, 16)] = srcs[j][pl.ds(g * 16, 16)] + coff

        def fire_gather(j, b):
            pltpu.async_copy(tab.at[srcs[j]], rows[b], gsems[b])

        def wait_gather(j, b):
            pltpu.make_async_copy(tab.at[srcs[j]], rows[b], gsems[b]).wait()

        def fire_scatter(b):
            pltpu.async_copy(rows[b], acc.at[dsts[b]], ssems[b], add=True)

        def wait_scatter(b):
            pltpu.make_async_copy(rows[b], acc.at[dsts[b]], ssems[b]).wait()

        def weight(j, b):
            def wbody(g, carry):
                e0g = g * np.int32(16)
                wv = wbufs[j][pl.ds(e0g, 16)]
                for jj in range(16):
                    wsc = wv[jj]
                    for kk in range(H // 16):
                        rows[b][e0g + jj, pl.ds(kk * 16, 16)] = (
                            rows[b][e0g + jj, pl.ds(kk * 16, 16)] * wsc)
                return carry
            lax.fori_loop(np.int32(0), np.int32(EC // 16), wbody, np.int32(0))

        def gen_iter(koff, j, b, wait_prev, fire_dst_nxt, fire_srcw4,
                     next_gather):
            if wait_prev:
                wait_scatter(b ^ 1)
            if fire_dst_nxt:
                fire_dst(koff + np.int32(EC), b ^ 1)
            wait_gather(j, b)
            wait_dst(b)
            weight(j, b)
            if fire_srcw4:
                fire_srcw(koff + np.int32(4 * EC), j)
            fire_scatter(b)
            if next_gather:
                jn = (j + 1) % 4
                wait_srcw(jn)
                adjust(jn)
                fire_gather(jn, b ^ 1)

        for j in range(4):
            fire_srcw(ebase + np.int32(j * EC), j)
        fire_dst(ebase, 0)
        wait_srcw(0)
        adjust(0)
        fire_gather(0, 0)
        for j in range(4):
            gen_iter(ebase + np.int32(j * EC), j, j % 2, j > 0, True, True,
                     True)

        def super_body(k4, carry):
            kbase = ebase + k4 * np.int32(4 * EC)
            for j in range(4):
                gen_iter(kbase + np.int32(j * EC), j, j % 2, True, True,
                         True, True)
            return carry
        lax.fori_loop(np.int32(1), np.int32(NCH4 - 1), super_body,
                      np.int32(0))

        kbase = ebase + np.int32((NCH4 - 1) * 4 * EC)
        for j in range(4):
            last = j == 3
            gen_iter(kbase + np.int32(j * EC), j, j % 2, True, not last,
                     False, not last)
        wait_scatter(1)

    def copyout_cur(dstref):
        for j in range(NRC):
            rr = r0 + np.int32(j * RC)
            pltpu.sync_copy(acc.at[pl.ds(rr, RC)], cb1)
            pltpu.sync_copy(cb1, dstref.at[pl.ds(coff + rr, RC)])

    def copyout_final():
        third = np.float32(1.0 / (LAYERS + 1))

        def addin(e, carry):
            for kk in range(H // 16):
                cb1[e, pl.ds(kk * 16, 16)] = (cb1[e, pl.ds(kk * 16, 16)]
                                              + cb2[e, pl.ds(kk * 16, 16)])
            return carry

        def addscale(e, carry):
            for kk in range(H // 16):
                cb1[e, pl.ds(kk * 16, 16)] = (cb1[e, pl.ds(kk * 16, 16)]
                                              + cb2[e, pl.ds(kk * 16, 16)]) * third
            return carry

        for j in range(NRC):
            rr = r0 + np.int32(j * RC)
            pltpu.sync_copy(acc.at[pl.ds(rr, RC)], cb1)
            pltpu.sync_copy(ego.at[pl.ds(coff + rr, RC)], cb2)
            lax.fori_loop(np.int32(0), np.int32(RC), addin, np.int32(0))
            pltpu.sync_copy(cur1.at[pl.ds(coff + rr, RC)], cb2)
            lax.fori_loop(np.int32(0), np.int32(RC), addscale, np.int32(0))
            pltpu.sync_copy(cb1, final.at[pl.ds(coff + rr, RC)])

    zero_cb1()
    zero_acc()
    plsc.subcore_barrier()
    edge_pass(ego)
    plsc.subcore_barrier()
    copyout_cur(cur1)
    zero_cb1()
    zero_acc()
    plsc.subcore_barrier()
    edge_pass(cur1)
    plsc.subcore_barrier()
    copyout_final()


def _propagate_sc(ego, src, dst, w):
    e = src.shape[0]
    ego_p = jnp.pad(ego, ((0, NP - N_NODES), (0, 0)))
    ego_h = ego_p.reshape(NP, 2, H).transpose(1, 0, 2).reshape(2 * NP, H)
    pad = EP - e
    ar = jnp.arange(pad, dtype=jnp.int32)
    src_p = jnp.concatenate([src, ar % N_NODES])
    dst_p = jnp.concatenate([dst, N_NODES + ar % (NP - N_NODES)])
    w_p = jnp.concatenate([w, jnp.zeros((pad,), jnp.float32)])

    mesh = plsc.VectorSubcoreMesh(core_axis_name="c", subcore_axis_name="s")
    out_final, _cur1 = pl.kernel(
        _prop_body,
        out_type=(jax.ShapeDtypeStruct((2 * NP, H), jnp.float32),
                  jax.ShapeDtypeStruct((2 * NP, H), jnp.float32)),
        mesh=mesh,
        scratch_types=[
            pltpu.VMEM_SHARED((NP, H), jnp.float32),
            pltpu.VMEM((EC, H), jnp.float32),
            pltpu.VMEM((EC, H), jnp.float32),
            pltpu.VMEM((EC,), jnp.int32),
            pltpu.VMEM((EC,), jnp.int32),
            pltpu.VMEM((EC,), jnp.int32),
            pltpu.VMEM((EC,), jnp.int32),
            pltpu.VMEM((EC,), jnp.float32),
            pltpu.VMEM((EC,), jnp.float32),
            pltpu.VMEM((EC,), jnp.float32),
            pltpu.VMEM((EC,), jnp.float32),
            pltpu.VMEM((EC,), jnp.int32),
            pltpu.VMEM((EC,), jnp.int32),
            pltpu.VMEM((RC, H), jnp.float32),
            pltpu.VMEM((RC, H), jnp.float32),
            pltpu.SemaphoreType.DMA,
            pltpu.SemaphoreType.DMA,
            pltpu.SemaphoreType.DMA,
            pltpu.SemaphoreType.DMA,
            pltpu.SemaphoreType.DMA,
            pltpu.SemaphoreType.DMA,
            pltpu.SemaphoreType.DMA,
            pltpu.SemaphoreType.DMA,
            pltpu.SemaphoreType.DMA,
            pltpu.SemaphoreType.DMA,
        ],
        compiler_params=pltpu.CompilerParams(use_tc_tiling_on_sc=False),
    )(ego_h, src_p, dst_p, w_p)
    emb = out_final.reshape(2, NP, H).transpose(1, 0, 2).reshape(NP, D)
    return emb[:N_NODES]


PB = 4096
PT = PB // (SC_NC * SC_NS)


def _dots_body(ua, ia, ub, ib, iua, iia, iub, iib, pos_a, pos_b,
               iu_v, ii_v, ru_v, ri_v, po_v, sem):
    c = lax.axis_index("c")
    s = lax.axis_index("s")
    wid = s * np.int32(SC_NC) + c
    base = wid * np.int32(PT)

    def one_domain(ut, it, iu, ii, pos):
        pltpu.sync_copy(iu.at[pl.ds(base, PT)], iu_v)
        pltpu.sync_copy(ii.at[pl.ds(base, PT)], ii_v)
        pltpu.async_copy(ut.at[iu_v], ru_v, sem)
        pltpu.async_copy(it.at[ii_v], ri_v, sem)
        pltpu.make_async_copy(ut.at[iu_v], ru_v, sem).wait()
        pltpu.make_async_copy(it.at[ii_v], ri_v, sem).wait()

        lanes = lax.broadcasted_iota(jnp.int32, (16,), 0)

        def pbody(g, carry):
            g0 = g * np.int32(16)
            vec = jnp.zeros((16,), jnp.float32)
            for j in range(16):
                p = g0 + j
                acc = ru_v[p, pl.ds(0, 16)] * ri_v[p, pl.ds(0, 16)]
                for kk in range(1, D // 16):
                    acc = acc + (ru_v[p, pl.ds(kk * 16, 16)]
                                 * ri_v[p, pl.ds(kk * 16, 16)])
                vec = jnp.where(lanes == j, jnp.sum(acc), vec)
            po_v[pl.ds(g0, 16)] = vec
            return carry
        lax.fori_loop(np.int32(0), np.int32(PT // 16), pbody, np.int32(0))
        pltpu.sync_copy(po_v, pos.at[pl.ds(base, PT)])

    one_domain(ua, ia, iua, iia, pos_a)
    one_domain(ub, ib, iub, iib, pos_b)


def _dots_sc(uA, iA, uB, iB, iua, iia, iub, iib):
    mesh = plsc.VectorSubcoreMesh(core_axis_name="c", subcore_axis_name="s")
    pos_a, pos_b = pl.kernel(
        _dots_body,
        out_type=(jax.ShapeDtypeStruct((PB,), jnp.float32),
                  jax.ShapeDtypeStruct((PB,), jnp.float32)),
        mesh=mesh,
        scratch_types=[
            pltpu.VMEM((PT,), jnp.int32),
            pltpu.VMEM((PT,), jnp.int32),
            pltpu.VMEM((PT, D), jnp.float32),
            pltpu.VMEM((PT, D), jnp.float32),
            pltpu.VMEM((PT,), jnp.float32),
            pltpu.SemaphoreType.DMA,
        ],
        compiler_params=pltpu.CompilerParams(needs_layout_passes=False),
    )(uA, iA, uB, iB, iua, iia, iub, iib)
    return pos_a, pos_b


def kernel(uEmb_a, iEmb_a, uEmb_b, iEmb_b, edge_weight_a, edge_weight_b,
           edge_index_a, edge_index_b, data_a, data_b):
    with _jcfg.enable_x64(False):
        return _kernel_32(uEmb_a, iEmb_a, uEmb_b, iEmb_b, edge_weight_a,
                          edge_weight_b, edge_index_a, edge_index_b,
                          data_a, data_b)


def _kernel_32(uEmb_a, iEmb_a, uEmb_b, iEmb_b, edge_weight_a, edge_weight_b,
               edge_index_a, edge_index_b, data_a, data_b):
    ei_a = edge_index_a.astype(jnp.int32)
    ei_b = edge_index_b.astype(jnp.int32)
    da = data_a.astype(jnp.int32)
    db = data_b.astype(jnp.int32)

    ego_a = jnp.concatenate([uEmb_a, iEmb_a], axis=0)
    ego_b = jnp.concatenate([uEmb_b, iEmb_b], axis=0)

    emb_a = _propagate_sc(ego_a, ei_a[0], ei_a[1], edge_weight_a)
    emb_b = _propagate_sc(ego_b, ei_b[0], ei_b[1], edge_weight_b)
    ua, iA = emb_a[:N_USERS], emb_a[N_USERS:]
    ub, iB = emb_b[:N_USERS], emb_b[N_USERS:]

    uB = _similarity_transfer(ub, ua)
    uA = _similarity_transfer(ua, ub)

    pos_a, pos_b = _dots_sc(uA, iA, uB, iB, da[0], da[1], db[0], db[1])
    return (pos_a, pos_b)


# speedup vs baseline: 1.4549x; 1.4549x over previous
import functools

import jax
import jax.numpy as jnp
import numpy as np
from jax import lax
from jax.experimental import pallas as pl
from jax.experimental.pallas import tpu as pltpu
from jax.experimental.pallas import tpu_sc as plsc
from jax._src import config as _jcfg

N_USERS = 15000
N_ITEMS = 10000
N_NODES = N_USERS + N_ITEMS
D = 128
LAYERS = 2

NEG_INF = np.float32(-1e30)
EPS = np.float32(1e-12)


def _flash_body(q_ref, k_ref, b_ref, o_ref, acc_ref, l_ref):
    j = pl.program_id(1)
    nj = pl.num_programs(1)

    @pl.when(j == 0)
    def _init():
        acc_ref[...] = jnp.zeros_like(acc_ref)
        l_ref[...] = jnp.zeros_like(l_ref)

    q = q_ref[...]
    qn = q * lax.reciprocal(jnp.sqrt(jnp.sum(q * q, axis=1, keepdims=True)) + EPS)
    k = k_ref[...]
    kn = k * lax.reciprocal(jnp.sqrt(jnp.sum(k * k, axis=1, keepdims=True)) + EPS)

    s = jax.lax.dot_general(qn.astype(jnp.bfloat16), kn.astype(jnp.bfloat16),
                            (((1,), (1,)), ((), ())),
                            preferred_element_type=jnp.float32)
    p = jnp.exp(s + b_ref[...])
    l_ref[...] = l_ref[...] + jnp.sum(p, axis=1, keepdims=True)
    pv = jax.lax.dot_general(p.astype(jnp.bfloat16), k.astype(jnp.bfloat16),
                             (((1,), (0,)), ((), ())),
                             preferred_element_type=jnp.float32)
    acc_ref[...] = acc_ref[...] + pv

    @pl.when(j == nj - 1)
    def _fin():
        o_ref[...] = q + acc_ref[...] * lax.reciprocal(l_ref[...])


def _similarity_transfer(target, auxi, bq=512, bk=1024):
    n, d = target.shape
    nk = auxi.shape[0]
    n_pad = (n + bq - 1) // bq * bq
    nk_pad = (nk + bk - 1) // bk * bk
    tq = jnp.pad(target, ((0, n_pad - n), (0, 0)))
    ta = jnp.pad(auxi, ((0, nk_pad - nk), (0, 0)))
    bias = jnp.asarray(
        np.where(np.arange(nk_pad) < nk, np.float32(-1.0),
                 np.float32(-1e30)).astype(np.float32).reshape(1, nk_pad))
    grid = (n_pad // bq, nk_pad // bk)
    out = pl.pallas_call(
        _flash_body,
        grid=grid,
        in_specs=[
            pl.BlockSpec((bq, d), lambda i, j: (i, np.int32(0))),
            pl.BlockSpec((bk, d), lambda i, j: (j, np.int32(0))),
            pl.BlockSpec((1, bk), lambda i, j: (np.int32(0), j)),
        ],
        out_specs=pl.BlockSpec((bq, d), lambda i, j: (i, np.int32(0))),
        out_shape=jax.ShapeDtypeStruct((n_pad, d), jnp.float32),
        scratch_shapes=[
            pltpu.VMEM((bq, d), jnp.float32),
            pltpu.VMEM((bq, 1), jnp.float32),
        ],
        compiler_params=pltpu.CompilerParams(
            dimension_semantics=("arbitrary", "arbitrary"),
        ),
    )(tq, ta, bias)
    return out[:n]


SC_NC = 2
SC_NS = 16
NP = 25088
H = D // 2
EP = 409600
ET = EP // SC_NS
EC = 128
NCH = ET // EC
RPT = NP // SC_NS
RC = 56
NRC = RPT // RC


def _prop_body(ego, srcr, dstr, wr, final, cur1,
               acc, rows0, rows1, src0, src1, src2, src3,
               w0, w1, w2, w3, dst0, dst1, cb1, cb2,
               isem0, isem1, isem2, isem3, dsem0, dsem1,
               gsem0, gsem1, ssem0, ssem1):
    c = lax.axis_index("c")
    s = lax.axis_index("s")
    r0 = s * np.int32(RPT)
    ebase = s * np.int32(ET)
    coff = c * np.int32(NP)

    def zero_cb1():
        def zb(e, carry):
            for kk in range(H // 16):
                cb1[e, pl.ds(kk * 16, 16)] = jnp.zeros((16,), jnp.float32)
            return carry
        lax.fori_loop(np.int32(0), np.int32(RC), zb, np.int32(0))

    def zero_acc():
        for j in range(NRC):
            pltpu.sync_copy(cb1, acc.at[pl.ds(r0 + np.int32(j * RC), RC)])

    def edge_pass(tab):
        srcs = (src0, src1, src2, src3)
        wbufs = (w0, w1, w2, w3)
        dsts = (dst0, dst1)
        rows = (rows0, rows1)
        isems = (isem0, isem1, isem2, isem3)
        dsems = (dsem0, dsem1)
        gsems = (gsem0, gsem1)
        ssems = (ssem0, ssem1)
        NCH4 = NCH // 4

        def fire_srcw(koff, j):
            pltpu.async_copy(srcr.at[pl.ds(koff, EC)], srcs[j], isems[j])
            pltpu.async_copy(wr.at[pl.ds(koff, EC)], wbufs[j], isems[j])

        def wait_srcw(j):
            pltpu.make_async_copy(srcr.at[pl.ds(ebase, EC)], srcs[j],
                                  isems[j]).wait()
            pltpu.make_async_copy(wr.at[pl.ds(ebase, EC)], wbufs[j],
                                  isems[j]).wait()

        def fire_dst(koff, b):
            pltpu.async_copy(dstr.at[pl.ds(koff, EC)], dsts[b], dsems[b])

        def wait_dst(b):
            pltpu.make_async_copy(dstr.at[pl.ds(ebase, EC)], dsts[b],
                                  dsems[b]).wait()

        def adjust(j):
            for g in range(EC // 16):
                srcs[j][pl.ds(g * 16, 16)] = srcs[j][pl.ds(g * 16, 16)] + coff

        def fire_gather(j, b):
            pltpu.async_copy(tab.at[srcs[j]], rows[b], gsems[b])

        def wait_gather(j, b):
            pltpu.make_async_copy(tab.at[srcs[j]], rows[b], gsems[b]).wait()

        def fire_scatter(b):
            pltpu.async_copy(rows[b], acc.at[dsts[b]], ssems[b], add=True)

        def wait_scatter(b):
            pltpu.make_async_copy(rows[b], acc.at[dsts[b]], ssems[b]).wait()

        def weight(j, b):
            def wbody(g, carry):
                e0g = g * np.int32(16)
                wv = wbufs[j][pl.ds(e0g, 16)]
                for jj in range(16):
                    wsc = wv[jj]
                    for kk in range(H // 16):
                        rows[b][e0g + jj, pl.ds(kk * 16, 16)] = (
                            rows[b][e0g + jj, pl.ds(kk * 16, 16)] * wsc)
                return carry
            lax.fori_loop(np.int32(0), np.int32(EC // 16), wbody, np.int32(0))

        def gen_iter(koff, j, b, wait_prev, fire_dst_nxt, fire_srcw4,
                     next_gather):
            if wait_prev:
                wait_scatter(b ^ 1)
            if fire_dst_nxt:
                fire_dst(koff + np.int32(EC), b ^ 1)
            wait_gather(j, b)
            wait_dst(b)
            weight(j, b)
            if fire_srcw4:
                fire_srcw(koff + np.int32(4 * EC), j)
            fire_scatter(b)
            if next_gather:
                jn = (j + 1) % 4
                wait_srcw(jn)
                adjust(jn)
                fire_gather(jn, b ^ 1)

        for j in range(4):
            fire_srcw(ebase + np.int32(j * EC), j)
        fire_dst(ebase, 0)
        wait_srcw(0)
        adjust(0)
        fire_gather(0, 0)
        for j in range(4):
            gen_iter(ebase + np.int32(j * EC), j, j % 2, j > 0, True, True,
                     True)

        def super_body(k4, carry):
            kbase = ebase + k4 * np.int32(4 * EC)
            for j in range(4):
                gen_iter(kbase + np.int32(j * EC), j, j % 2, True, True,
                         True, True)
            return carry
        lax.fori_loop(np.int32(1), np.int32(NCH4 - 1), super_body,
                      np.int32(0))

        kbase = ebase + np.int32((NCH4 - 1) * 4 * EC)
        for j in range(4):
            last = j == 3
            gen_iter(kbase + np.int32(j * EC), j, j % 2, True, not last,
                     False, not last)
        wait_scatter(1)

    def copyout_cur(dstref):
        for j in range(NRC):
            rr = r0 + np.int32(j * RC)
            pltpu.sync_copy(acc.at[pl.ds(rr, RC)], cb1)
            pltpu.sync_copy(cb1, dstref.at[pl.ds(coff + rr, RC)])

    def copyout_final():
        third = np.float32(1.0 / (LAYERS + 1))

        def addin(e, carry):
            for kk in range(H // 16):
                cb1[e, pl.ds(kk * 16, 16)] = (cb1[e, pl.ds(kk * 16, 16)]
                                              + cb2[e, pl.ds(kk * 16, 16)])
            return carry

        def addscale(e, carry):
            for kk in range(H // 16):
                cb1[e, pl.ds(kk * 16, 16)] = (cb1[e, pl.ds(kk * 16, 16)]
                                              + cb2[e, pl.ds(kk * 16, 16)]) * third
            return carry

        for j in range(NRC):
            rr = r0 + np.int32(j * RC)
            pltpu.sync_copy(acc.at[pl.ds(rr, RC)], cb1)
            pltpu.sync_copy(ego.at[pl.ds(coff + rr, RC)], cb2)
            lax.fori_loop(np.int32(0), np.int32(RC), addin, np.int32(0))
            pltpu.sync_copy(cur1.at[pl.ds(coff + rr, RC)], cb2)
            lax.fori_loop(np.int32(0), np.int32(RC), addscale, np.int32(0))
            pltpu.sync_copy(cb1, final.at[pl.ds(coff + rr, RC)])

    zero_cb1()
    zero_acc()
    plsc.subcore_barrier()
    edge_pass(ego)
    plsc.subcore_barrier()
    copyout_cur(cur1)
    zero_cb1()
    zero_acc()
    plsc.subcore_barrier()
    edge_pass(cur1)
    plsc.subcore_barrier()
    copyout_final()


def _propagate_sc(ego, src, dst, w):
    e = src.shape[0]
    ego_p = jnp.pad(ego, ((0, NP - N_NODES), (0, 0)))
    ego_h = ego_p.reshape(NP, 2, H).transpose(1, 0, 2).reshape(2 * NP, H)
    pad = EP - e
    ar = jnp.arange(pad, dtype=jnp.int32)
    src_p = jnp.concatenate([src, ar % N_NODES])
    dst_p = jnp.concatenate([dst, N_NODES + ar % (NP - N_NODES)])
    w_p = jnp.concatenate([w, jnp.zeros((pad,), jnp.float32)])

    mesh = plsc.VectorSubcoreMesh(core_axis_name="c", subcore_axis_name="s")
    out_final, _cur1 = pl.kernel(
        _prop_body,
        out_type=(jax.ShapeDtypeStruct((2 * NP, H), jnp.float32),
                  jax.ShapeDtypeStruct((2 * NP, H), jnp.float32)),
        mesh=mesh,
        scratch_types=[
            pltpu.VMEM_SHARED((NP, H), jnp.float32),
            pltpu.VMEM((EC, H), jnp.float32),
            pltpu.VMEM((EC, H), jnp.float32),
            pltpu.VMEM((EC,), jnp.int32),
            pltpu.VMEM((EC,), jnp.int32),
            pltpu.VMEM((EC,), jnp.int32),
            pltpu.VMEM((EC,), jnp.int32),
            pltpu.VMEM((EC,), jnp.float32),
            pltpu.VMEM((EC,), jnp.float32),
            pltpu.VMEM((EC,), jnp.float32),
            pltpu.VMEM((EC,), jnp.float32),
            pltpu.VMEM((EC,), jnp.int32),
            pltpu.VMEM((EC,), jnp.int32),
            pltpu.VMEM((RC, H), jnp.float32),
            pltpu.VMEM((RC, H), jnp.float32),
            pltpu.SemaphoreType.DMA,
            pltpu.SemaphoreType.DMA,
            pltpu.SemaphoreType.DMA,
            pltpu.SemaphoreType.DMA,
            pltpu.SemaphoreType.DMA,
            pltpu.SemaphoreType.DMA,
            pltpu.SemaphoreType.DMA,
            pltpu.SemaphoreType.DMA,
            pltpu.SemaphoreType.DMA,
            pltpu.SemaphoreType.DMA,
        ],
        compiler_params=pltpu.CompilerParams(use_tc_tiling_on_sc=False),
    )(ego_h, src_p, dst_p, w_p)
    emb = out_final.reshape(2, NP, H).transpose(1, 0, 2).reshape(NP, D)
    return emb[:N_NODES]


PB = 4096
PT = PB // (SC_NC * SC_NS)


def _dots_body(ua, ia, ub, ib, iua, iia, iub, iib, pos_a, pos_b,
               iu_v, ii_v, ru_v, ri_v, po_v, sem):
    c = lax.axis_index("c")
    s = lax.axis_index("s")
    wid = s * np.int32(SC_NC) + c
    base = wid * np.int32(PT)

    def one_domain(ut, it, iu, ii, pos):
        pltpu.sync_copy(iu.at[pl.ds(base, PT)], iu_v)
        pltpu.sync_copy(ii.at[pl.ds(base, PT)], ii_v)
        pltpu.async_copy(ut.at[iu_v], ru_v, sem)
        pltpu.async_copy(it.at[ii_v], ri_v, sem)
        pltpu.make_async_copy(ut.at[iu_v], ru_v, sem).wait()
        pltpu.make_async_copy(it.at[ii_v], ri_v, sem).wait()

        lanes = lax.broadcasted_iota(jnp.int32, (16,), 0)

        def pbody(g, carry):
            g0 = g * np.int32(16)
            vec = jnp.zeros((16,), jnp.float32)
            for j in range(16):
                p = g0 + j
                acc = ru_v[p, pl.ds(0, 16)] * ri_v[p, pl.ds(0, 16)]
                for kk in range(1, D // 16):
                    acc = acc + (ru_v[p, pl.ds(kk * 16, 16)]
                                 * ri_v[p, pl.ds(kk * 16, 16)])
                vec = jnp.where(lanes == j, jnp.sum(acc), vec)
            po_v[pl.ds(g0, 16)] = vec
            return carry
        lax.fori_loop(np.int32(0), np.int32(PT // 16), pbody, np.int32(0))
        pltpu.sync_copy(po_v, pos.at[pl.ds(base, PT)])

    one_domain(ua, ia, iua, iia, pos_a)
    one_domain(ub, ib, iub, iib, pos_b)


def _dots_sc(uA, iA, uB, iB, iua, iia, iub, iib):
    mesh = plsc.VectorSubcoreMesh(core_axis_name="c", subcore_axis_name="s")
    pos_a, pos_b = pl.kernel(
        _dots_body,
        out_type=(jax.ShapeDtypeStruct((PB,), jnp.float32),
                  jax.ShapeDtypeStruct((PB,), jnp.float32)),
        mesh=mesh,
        scratch_types=[
            pltpu.VMEM((PT,), jnp.int32),
            pltpu.VMEM((PT,), jnp.int32),
            pltpu.VMEM((PT, D), jnp.float32),
            pltpu.VMEM((PT, D), jnp.float32),
            pltpu.VMEM((PT,), jnp.float32),
            pltpu.SemaphoreType.DMA,
        ],
        compiler_params=pltpu.CompilerParams(needs_layout_passes=False),
    )(uA, iA, uB, iB, iua, iia, iub, iib)
    return pos_a, pos_b


def kernel(uEmb_a, iEmb_a, uEmb_b, iEmb_b, edge_weight_a, edge_weight_b,
           edge_index_a, edge_index_b, data_a, data_b):
    with _jcfg.enable_x64(False):
        return _kernel_32(uEmb_a, iEmb_a, uEmb_b, iEmb_b, edge_weight_a,
                          edge_weight_b, edge_index_a, edge_index_b,
                          data_a, data_b)


def _kernel_32(uEmb_a, iEmb_a, uEmb_b, iEmb_b, edge_weight_a, edge_weight_b,
               edge_index_a, edge_index_b, data_a, data_b):
    ei_a = edge_index_a.astype(jnp.int32)
    ei_b = edge_index_b.astype(jnp.int32)
    da = data_a.astype(jnp.int32)
    db = data_b.astype(jnp.int32)

    ego_a = jnp.concatenate([uEmb_a, iEmb_a], axis=0)
    ego_b = jnp.concatenate([uEmb_b, iEmb_b], axis=0)

    emb_a = _propagate_sc(ego_a, ei_a[0], ei_a[1], edge_weight_a)
    emb_b = _propagate_sc(ego_b, ei_b[0], ei_b[1], edge_weight_b)
    ua, iA = emb_a[:N_USERS], emb_a[N_USERS:]
    ub, iB = emb_b[:N_USERS], emb_b[N_USERS:]

    uB = _similarity_transfer(ub, ua)
    uA = _similarity_transfer(ua, ub)

    pos_a, pos_b = _dots_sc(uA, iA, uB, iB, da[0], da[1], db[0], db[1])
    return (pos_a, pos_b)


# speedup vs baseline: 2.1517x; 1.4790x over previous
import functools

import jax
import jax.numpy as jnp
import numpy as np
from jax import lax
from jax.experimental import pallas as pl
from jax.experimental.pallas import tpu as pltpu
from jax.experimental.pallas import tpu_sc as plsc
from jax._src import config as _jcfg

N_USERS = 15000
N_ITEMS = 10000
N_NODES = N_USERS + N_ITEMS
D = 128
LAYERS = 2

NEG_INF = np.float32(-1e30)
EPS = np.float32(1e-12)


def _flash_body(q_ref, k_ref, b_ref, o_ref, acc_ref, l_ref):
    j = pl.program_id(1)
    nj = pl.num_programs(1)

    @pl.when(j == 0)
    def _init():
        acc_ref[...] = jnp.zeros_like(acc_ref)
        l_ref[...] = jnp.zeros_like(l_ref)

    q = q_ref[...]
    qn = q * lax.reciprocal(jnp.sqrt(jnp.sum(q * q, axis=1, keepdims=True)) + EPS)
    k = k_ref[...]
    kn = k * lax.reciprocal(jnp.sqrt(jnp.sum(k * k, axis=1, keepdims=True)) + EPS)

    s = jax.lax.dot_general(qn.astype(jnp.bfloat16), kn.astype(jnp.bfloat16),
                            (((1,), (1,)), ((), ())),
                            preferred_element_type=jnp.float32)
    p = jnp.exp(s + b_ref[...])
    l_ref[...] = l_ref[...] + jnp.sum(p, axis=1, keepdims=True)
    pv = jax.lax.dot_general(p.astype(jnp.bfloat16), k.astype(jnp.bfloat16),
                             (((1,), (0,)), ((), ())),
                             preferred_element_type=jnp.float32)
    acc_ref[...] = acc_ref[...] + pv

    @pl.when(j == nj - 1)
    def _fin():
        o_ref[...] = q + acc_ref[...] * lax.reciprocal(l_ref[...])


def _similarity_transfer(target, auxi, bq=512, bk=1024):
    n, d = target.shape
    nk = auxi.shape[0]
    n_pad = (n + bq - 1) // bq * bq
    nk_pad = (nk + bk - 1) // bk * bk
    tq = jnp.pad(target, ((0, n_pad - n), (0, 0)))
    ta = jnp.pad(auxi, ((0, nk_pad - nk), (0, 0)))
    bias = jnp.asarray(
        np.where(np.arange(nk_pad) < nk, np.float32(-1.0),
                 np.float32(-1e30)).astype(np.float32).reshape(1, nk_pad))
    grid = (n_pad // bq, nk_pad // bk)
    out = pl.pallas_call(
        _flash_body,
        grid=grid,
        in_specs=[
            pl.BlockSpec((bq, d), lambda i, j: (i, np.int32(0))),
            pl.BlockSpec((bk, d), lambda i, j: (j, np.int32(0))),
            pl.BlockSpec((1, bk), lambda i, j: (np.int32(0), j)),
        ],
        out_specs=pl.BlockSpec((bq, d), lambda i, j: (i, np.int32(0))),
        out_shape=jax.ShapeDtypeStruct((n_pad, d), jnp.float32),
        scratch_shapes=[
            pltpu.VMEM((bq, d), jnp.float32),
            pltpu.VMEM((bq, 1), jnp.float32),
        ],
        compiler_params=pltpu.CompilerParams(
            dimension_semantics=("arbitrary", "arbitrary"),
        ),
    )(tq, ta, bias)
    return out[:n]


SC_NC = 2
SC_NS = 16
NP = 25088
H = D // 2
EP = 409600
ET = EP // SC_NS
EC = 128
NCH = ET // EC
RPT = NP // SC_NS
RC = 28
NRC = RPT // RC


def _prop_body(ego, srcr, dstr, wr, final, cur1,
               acc, rows0, rows1, msg_v, src0, src1,
               w0, w1, dst0, dst1, cb1, cb2,
               isem0, isem1, dsem0, dsem1,
               gsem0, gsem1, ssem0, ssem1):
    c = lax.axis_index("c")
    s = lax.axis_index("s")
    r0 = s * np.int32(RPT)
    ebase = s * np.int32(ET)
    coff = c * np.int32(NP)

    def zero_cb1():
        def zb(e, carry):
            for kk in range(H // 16):
                cb1[e, pl.ds(kk * 16, 16)] = jnp.zeros((16,), jnp.float32)
            return carry
        lax.fori_loop(np.int32(0), np.int32(RC), zb, np.int32(0))

    def zero_acc():
        for j in range(NRC):
            pltpu.sync_copy(cb1, acc.at[pl.ds(r0 + np.int32(j * RC), RC)])

    def edge_pass(tab):
        srcs = (src0, src1)
        wbufs = (w0, w1)
        dsts = (dst0, dst1)
        rows = (rows0, rows1)
        isems = (isem0, isem1)
        dsems = (dsem0, dsem1)
        gsems = (gsem0, gsem1)
        ssems = (ssem0, ssem1)
        NCH2 = NCH // 2

        def fire_srcw(koff, j):
            pltpu.async_copy(srcr.at[pl.ds(koff, EC)], srcs[j], isems[j])
            pltpu.async_copy(wr.at[pl.ds(koff, EC)], wbufs[j], isems[j])

        def wait_srcw(j):
            pltpu.make_async_copy(srcr.at[pl.ds(ebase, EC)], srcs[j],
                                  isems[j]).wait()
            pltpu.make_async_copy(wr.at[pl.ds(ebase, EC)], wbufs[j],
                                  isems[j]).wait()

        def fire_dst(koff, b):
            pltpu.async_copy(dstr.at[pl.ds(koff, EC)], dsts[b], dsems[b])

        def wait_dst(b):
            pltpu.make_async_copy(dstr.at[pl.ds(ebase, EC)], dsts[b],
                                  dsems[b]).wait()

        def adjust(j):
            for g in range(EC // 16):
                srcs[j][pl.ds(g * 16, 16)] = srcs[j][pl.ds(g * 16, 16)] + coff

        def fire_gather(j, b):
            pltpu.async_copy(tab.at[srcs[j]], rows[b], gsems[b])

        def wait_gather(j, b):
            pltpu.make_async_copy(tab.at[srcs[j]], rows[b], gsems[b]).wait()

        def fire_scatter(b):
            pltpu.async_copy(msg_v, acc.at[dsts[b]], ssems[b], add=True)

        def wait_scatter(b):
            pltpu.make_async_copy(msg_v, acc.at[dsts[b]], ssems[b]).wait()

        def weight(j, b):
            def wbody(g, carry):
                e0g = g * np.int32(16)
                wv = wbufs[j][pl.ds(e0g, 16)]
                for jj in range(16):
                    wsc = wv[jj]
                    for kk in range(H // 16):
                        msg_v[e0g + jj, pl.ds(kk * 16, 16)] = (
                            rows[b][e0g + jj, pl.ds(kk * 16, 16)] * wsc)
                return carry
            lax.fori_loop(np.int32(0), np.int32(EC // 16), wbody, np.int32(0))

        def gen_iter(koff, b, wait_prev, fire_dst_nxt, fire_srcw2,
                     next_gather):
            if wait_prev:
                wait_scatter(b ^ 1)
            if fire_dst_nxt:
                fire_dst(koff + np.int32(EC), b ^ 1)
            wait_gather(b, b)
            wait_dst(b)
            weight(b, b)
            if fire_srcw2:
                fire_srcw(koff + np.int32(2 * EC), b)
            fire_scatter(b)
            if next_gather:
                wait_srcw(b ^ 1)
                adjust(b ^ 1)
                fire_gather(b ^ 1, b ^ 1)

        fire_srcw(ebase, 0)
        fire_srcw(ebase + np.int32(EC), 1)
        fire_dst(ebase, 0)
        wait_srcw(0)
        adjust(0)
        fire_gather(0, 0)
        gen_iter(ebase, 0, False, True, True, True)
        gen_iter(ebase + np.int32(EC), 1, True, True, True, True)

        def super_body(k2, carry):
            kbase = ebase + k2 * np.int32(2 * EC)
            gen_iter(kbase, 0, True, True, True, True)
            gen_iter(kbase + np.int32(EC), 1, True, True, True, True)
            return carry
        lax.fori_loop(np.int32(1), np.int32(NCH2 - 1), super_body,
                      np.int32(0))

        kbase = ebase + np.int32((NCH2 - 1) * 2 * EC)
        gen_iter(kbase, 0, True, True, False, True)
        gen_iter(kbase + np.int32(EC), 1, True, False, False, False)
        wait_scatter(1)

    def copyout_cur(dstref):
        for j in range(NRC):
            rr = r0 + np.int32(j * RC)
            pltpu.sync_copy(acc.at[pl.ds(rr, RC)], cb1)
            pltpu.sync_copy(cb1, dstref.at[pl.ds(coff + rr, RC)])

    def copyout_final():
        third = np.float32(1.0 / (LAYERS + 1))

        def addin(e, carry):
            for kk in range(H // 16):
                cb1[e, pl.ds(kk * 16, 16)] = (cb1[e, pl.ds(kk * 16, 16)]
                                              + cb2[e, pl.ds(kk * 16, 16)])
            return carry

        def addscale(e, carry):
            for kk in range(H // 16):
                cb1[e, pl.ds(kk * 16, 16)] = (cb1[e, pl.ds(kk * 16, 16)]
                                              + cb2[e, pl.ds(kk * 16, 16)]) * third
            return carry

        for j in range(NRC):
            rr = r0 + np.int32(j * RC)
            pltpu.sync_copy(acc.at[pl.ds(rr, RC)], cb1)
            pltpu.sync_copy(ego.at[pl.ds(coff + rr, RC)], cb2)
            lax.fori_loop(np.int32(0), np.int32(RC), addin, np.int32(0))
            pltpu.sync_copy(cur1.at[pl.ds(coff + rr, RC)], cb2)
            lax.fori_loop(np.int32(0), np.int32(RC), addscale, np.int32(0))
            pltpu.sync_copy(cb1, final.at[pl.ds(coff + rr, RC)])

    zero_cb1()
    zero_acc()
    plsc.subcore_barrier()
    edge_pass(ego)
    plsc.subcore_barrier()
    copyout_cur(cur1)
    zero_cb1()
    zero_acc()
    plsc.subcore_barrier()
    edge_pass(cur1)
    plsc.subcore_barrier()
    copyout_final()


def _propagate_sc(ego, src, dst, w):
    e = src.shape[0]
    ego_p = jnp.pad(ego, ((0, NP - N_NODES), (0, 0)))
    ego_h = ego_p.reshape(NP, 2, H).transpose(1, 0, 2).reshape(2 * NP, H)
    pad = EP - e
    ar = jnp.arange(pad, dtype=jnp.int32)
    src_p = jnp.concatenate([src, ar % N_NODES])
    dst_p = jnp.concatenate([dst, N_NODES + ar % (NP - N_NODES)])
    w_p = jnp.concatenate([w, jnp.zeros((pad,), jnp.float32)])

    mesh = plsc.VectorSubcoreMesh(core_axis_name="c", subcore_axis_name="s")
    out_final, _cur1 = pl.kernel(
        _prop_body,
        out_type=(jax.ShapeDtypeStruct((2 * NP, H), jnp.float32),
                  jax.ShapeDtypeStruct((2 * NP, H), jnp.float32)),
        mesh=mesh,
        scratch_types=[
            pltpu.VMEM_SHARED((NP, H), jnp.float32),
            pltpu.VMEM((EC, H), jnp.float32),
            pltpu.VMEM((EC, H), jnp.float32),
            pltpu.VMEM((EC, H), jnp.float32),
            pltpu.VMEM((EC,), jnp.int32),
            pltpu.VMEM((EC,), jnp.int32),
            pltpu.VMEM((EC,), jnp.float32),
            pltpu.VMEM((EC,), jnp.float32),
            pltpu.VMEM((EC,), jnp.int32),
            pltpu.VMEM((EC,), jnp.int32),
            pltpu.VMEM((RC, H), jnp.float32),
            pltpu.VMEM((RC, H), jnp.float32),
            pltpu.SemaphoreType.DMA,
            pltpu.SemaphoreType.DMA,
            pltpu.SemaphoreType.DMA,
            pltpu.SemaphoreType.DMA,
            pltpu.SemaphoreType.DMA,
            pltpu.SemaphoreType.DMA,
            pltpu.SemaphoreType.DMA,
            pltpu.SemaphoreType.DMA,
        ],
        compiler_params=pltpu.CompilerParams(use_tc_tiling_on_sc=False),
    )(ego_h, src_p, dst_p, w_p)
    emb = out_final.reshape(2, NP, H).transpose(1, 0, 2).reshape(NP, D)
    return emb[:N_NODES]


PB = 4096
PT = PB // (SC_NC * SC_NS)


def _dots_body(ua, ia, ub, ib, iua, iia, iub, iib, pos_a, pos_b,
               iu_v, ii_v, ru_v, ri_v, po_v, sem):
    c = lax.axis_index("c")
    s = lax.axis_index("s")
    wid = s * np.int32(SC_NC) + c
    base = wid * np.int32(PT)

    def one_domain(ut, it, iu, ii, pos):
        pltpu.sync_copy(iu.at[pl.ds(base, PT)], iu_v)
        pltpu.sync_copy(ii.at[pl.ds(base, PT)], ii_v)
        pltpu.async_copy(ut.at[iu_v], ru_v, sem)
        pltpu.async_copy(it.at[ii_v], ri_v, sem)
        pltpu.make_async_copy(ut.at[iu_v], ru_v, sem).wait()
        pltpu.make_async_copy(it.at[ii_v], ri_v, sem).wait()

        lanes = lax.broadcasted_iota(jnp.int32, (16,), 0)

        def pbody(g, carry):
            g0 = g * np.int32(16)
            vec = jnp.zeros((16,), jnp.float32)
            for j in range(16):
                p = g0 + j
                acc = ru_v[p, pl.ds(0, 16)] * ri_v[p, pl.ds(0, 16)]
                for kk in range(1, D // 16):
                    acc = acc + (ru_v[p, pl.ds(kk * 16, 16)]
                                 * ri_v[p, pl.ds(kk * 16, 16)])
                vec = jnp.where(lanes == j, jnp.sum(acc), vec)
            po_v[pl.ds(g0, 16)] = vec
            return carry
        lax.fori_loop(np.int32(0), np.int32(PT // 16), pbody, np.int32(0))
        pltpu.sync_copy(po_v, pos.at[pl.ds(base, PT)])

    one_domain(ua, ia, iua, iia, pos_a)
    one_domain(ub, ib, iub, iib, pos_b)


def _dots_sc(uA, iA, uB, iB, iua, iia, iub, iib):
    mesh = plsc.VectorSubcoreMesh(core_axis_name="c", subcore_axis_name="s")
    pos_a, pos_b = pl.kernel(
        _dots_body,
        out_type=(jax.ShapeDtypeStruct((PB,), jnp.float32),
                  jax.ShapeDtypeStruct((PB,), jnp.float32)),
        mesh=mesh,
        scratch_types=[
            pltpu.VMEM((PT,), jnp.int32),
            pltpu.VMEM((PT,), jnp.int32),
            pltpu.VMEM((PT, D), jnp.float32),
            pltpu.VMEM((PT, D), jnp.float32),
            pltpu.VMEM((PT,), jnp.float32),
            pltpu.SemaphoreType.DMA,
        ],
        compiler_params=pltpu.CompilerParams(needs_layout_passes=False),
    )(uA, iA, uB, iB, iua, iia, iub, iib)
    return pos_a, pos_b


def kernel(uEmb_a, iEmb_a, uEmb_b, iEmb_b, edge_weight_a, edge_weight_b,
           edge_index_a, edge_index_b, data_a, data_b):
    with _jcfg.enable_x64(False):
        return _kernel_32(uEmb_a, iEmb_a, uEmb_b, iEmb_b, edge_weight_a,
                          edge_weight_b, edge_index_a, edge_index_b,
                          data_a, data_b)


def _kernel_32(uEmb_a, iEmb_a, uEmb_b, iEmb_b, edge_weight_a, edge_weight_b,
               edge_index_a, edge_index_b, data_a, data_b):
    ei_a = edge_index_a.astype(jnp.int32)
    ei_b = edge_index_b.astype(jnp.int32)
    da = data_a.astype(jnp.int32)
    db = data_b.astype(jnp.int32)

    ego_a = jnp.concatenate([uEmb_a, iEmb_a], axis=0)
    ego_b = jnp.concatenate([uEmb_b, iEmb_b], axis=0)

    emb_a = _propagate_sc(ego_a, ei_a[0], ei_a[1], edge_weight_a)
    emb_b = _propagate_sc(ego_b, ei_b[0], ei_b[1], edge_weight_b)
    ua, iA = emb_a[:N_USERS], emb_a[N_USERS:]
    ub, iB = emb_b[:N_USERS], emb_b[N_USERS:]

    uB = _similarity_transfer(ub, ua)
    uA = _similarity_transfer(ua, ub)

    pos_a, pos_b = _dots_sc(uA, iA, uB, iB, da[0], da[1], db[0], db[1])
    return (pos_a, pos_b)


# speedup vs baseline: 2.2601x; 1.0504x over previous
import functools

import jax
import jax.numpy as jnp
import numpy as np
from jax import lax
from jax.experimental import pallas as pl
from jax.experimental.pallas import tpu as pltpu
from jax.experimental.pallas import tpu_sc as plsc
from jax._src import config as _jcfg

N_USERS = 15000
N_ITEMS = 10000
N_NODES = N_USERS + N_ITEMS
D = 128
LAYERS = 2

NEG_INF = np.float32(-1e30)
EPS = np.float32(1e-12)


def _flash_body(q_ref, k_ref, b_ref, o_ref, acc_ref, l_ref, qn_ref,
                kn_ref, kb_ref):
    i = pl.program_id(0)
    j = pl.program_id(1)
    nj = pl.num_programs(1)
    bk = k_ref.shape[0]

    @pl.when(j == 0)
    def _init():
        acc_ref[...] = jnp.zeros_like(acc_ref)
        l_ref[...] = jnp.zeros_like(l_ref)
        q = q_ref[...]
        qn = q * lax.reciprocal(
            jnp.sqrt(jnp.sum(q * q, axis=1, keepdims=True)) + EPS)
        qn_ref[...] = qn.astype(jnp.bfloat16)

    @pl.when(i == 0)
    def _knorm():
        k = k_ref[...]
        kn = k * lax.reciprocal(
            jnp.sqrt(jnp.sum(k * k, axis=1, keepdims=True)) + EPS)
        kn_ref[pl.ds(j * bk, bk), :] = kn.astype(jnp.bfloat16)
        kb_ref[pl.ds(j * bk, bk), :] = k.astype(jnp.bfloat16)

    s = jax.lax.dot_general(qn_ref[...], kn_ref[pl.ds(j * bk, bk), :],
                            (((1,), (1,)), ((), ())),
                            preferred_element_type=jnp.float32)
    p = jnp.exp(s + b_ref[...])
    l_ref[...] = l_ref[...] + jnp.sum(p, axis=1, keepdims=True)
    pv = jax.lax.dot_general(p.astype(jnp.bfloat16),
                             kb_ref[pl.ds(j * bk, bk), :],
                             (((1,), (0,)), ((), ())),
                             preferred_element_type=jnp.float32)
    acc_ref[...] = acc_ref[...] + pv

    @pl.when(j == nj - 1)
    def _fin():
        o_ref[...] = q_ref[...] + acc_ref[...] * lax.reciprocal(l_ref[...])


def _similarity_transfer(target, auxi, bq=512, bk=1024):
    n, d = target.shape
    nk = auxi.shape[0]
    n_pad = (n + bq - 1) // bq * bq
    nk_pad = (nk + bk - 1) // bk * bk
    tq = jnp.pad(target, ((0, n_pad - n), (0, 0)))
    ta = jnp.pad(auxi, ((0, nk_pad - nk), (0, 0)))
    bias = jnp.asarray(
        np.where(np.arange(nk_pad) < nk, np.float32(-1.0),
                 np.float32(-1e30)).astype(np.float32).reshape(1, nk_pad))
    grid = (n_pad // bq, nk_pad // bk)
    out = pl.pallas_call(
        _flash_body,
        grid=grid,
        in_specs=[
            pl.BlockSpec((bq, d), lambda i, j: (i, np.int32(0))),
            pl.BlockSpec((bk, d), lambda i, j: (j, np.int32(0))),
            pl.BlockSpec((1, bk), lambda i, j: (np.int32(0), j)),
        ],
        out_specs=pl.BlockSpec((bq, d), lambda i, j: (i, np.int32(0))),
        out_shape=jax.ShapeDtypeStruct((n_pad, d), jnp.float32),
        scratch_shapes=[
            pltpu.VMEM((bq, d), jnp.float32),
            pltpu.VMEM((bq, 1), jnp.float32),
            pltpu.VMEM((bq, d), jnp.bfloat16),
            pltpu.VMEM((nk_pad, d), jnp.bfloat16),
            pltpu.VMEM((nk_pad, d), jnp.bfloat16),
        ],
        compiler_params=pltpu.CompilerParams(
            dimension_semantics=("arbitrary", "arbitrary"),
        ),
    )(tq, ta, bias)
    return out[:n]


SC_NC = 2
SC_NS = 16
NP = 25088
H = D // 2
EP = 409600
ET = EP // SC_NS
EC = 128
NCH = ET // EC
RPT = NP // SC_NS
RC = 28
NRC = RPT // RC


def _prop_body(ego, srcr, dstr, wr, final, cur1,
               acc, rows0, rows1, msg_v, src0, src1,
               w0, w1, dst0, dst1, cb1, cb2,
               isem0, isem1, dsem0, dsem1,
               gsem0, gsem1, ssem0, ssem1):
    c = lax.axis_index("c")
    s = lax.axis_index("s")
    r0 = s * np.int32(RPT)
    ebase = s * np.int32(ET)
    coff = c * np.int32(NP)

    def zero_cb1():
        def zb(e, carry):
            for kk in range(H // 16):
                cb1[e, pl.ds(kk * 16, 16)] = jnp.zeros((16,), jnp.float32)
            return carry
        lax.fori_loop(np.int32(0), np.int32(RC), zb, np.int32(0))

    def zero_acc():
        for j in range(NRC):
            pltpu.sync_copy(cb1, acc.at[pl.ds(r0 + np.int32(j * RC), RC)])

    def edge_pass(tab):
        srcs = (src0, src1)
        wbufs = (w0, w1)
        dsts = (dst0, dst1)
        rows = (rows0, rows1)
        isems = (isem0, isem1)
        dsems = (dsem0, dsem1)
        gsems = (gsem0, gsem1)
        ssems = (ssem0, ssem1)
        NCH2 = NCH // 2

        def fire_srcw(koff, j):
            pltpu.async_copy(srcr.at[pl.ds(koff, EC)], srcs[j], isems[j])
            pltpu.async_copy(wr.at[pl.ds(koff, EC)], wbufs[j], isems[j])

        def wait_srcw(j):
            pltpu.make_async_copy(srcr.at[pl.ds(ebase, EC)], srcs[j],
                                  isems[j]).wait()
            pltpu.make_async_copy(wr.at[pl.ds(ebase, EC)], wbufs[j],
                                  isems[j]).wait()

        def fire_dst(koff, b):
            pltpu.async_copy(dstr.at[pl.ds(koff, EC)], dsts[b], dsems[b])

        def wait_dst(b):
            pltpu.make_async_copy(dstr.at[pl.ds(ebase, EC)], dsts[b],
                                  dsems[b]).wait()

        def adjust(j):
            for g in range(EC // 16):
                srcs[j][pl.ds(g * 16, 16)] = srcs[j][pl.ds(g * 16, 16)] + coff

        def fire_gather(j, b):
            pltpu.async_copy(tab.at[srcs[j]], rows[b], gsems[b])

        def wait_gather(j, b):
            pltpu.make_async_copy(tab.at[srcs[j]], rows[b], gsems[b]).wait()

        def fire_scatter(b):
            pltpu.async_copy(msg_v, acc.at[dsts[b]], ssems[b], add=True)

        def wait_scatter(b):
            pltpu.make_async_copy(msg_v, acc.at[dsts[b]], ssems[b]).wait()

        def weight(j, b):
            def wbody(g, carry):
                e0g = g * np.int32(16)
                wv = wbufs[j][pl.ds(e0g, 16)]
                for jj in range(16):
                    wsc = wv[jj]
                    for kk in range(H // 16):
                        msg_v[e0g + jj, pl.ds(kk * 16, 16)] = (
                            rows[b][e0g + jj, pl.ds(kk * 16, 16)] * wsc)
                return carry
            lax.fori_loop(np.int32(0), np.int32(EC // 16), wbody, np.int32(0))

        def gen_iter(koff, b, wait_prev, fire_dst_nxt, fire_srcw2,
                     next_gather):
            if wait_prev:
                wait_scatter(b ^ 1)
            if fire_dst_nxt:
                fire_dst(koff + np.int32(EC), b ^ 1)
            wait_gather(b, b)
            wait_dst(b)
            weight(b, b)
            if fire_srcw2:
                fire_srcw(koff + np.int32(2 * EC), b)
            fire_scatter(b)
            if next_gather:
                wait_srcw(b ^ 1)
                adjust(b ^ 1)
                fire_gather(b ^ 1, b ^ 1)

        fire_srcw(ebase, 0)
        fire_srcw(ebase + np.int32(EC), 1)
        fire_dst(ebase, 0)
        wait_srcw(0)
        adjust(0)
        fire_gather(0, 0)
        gen_iter(ebase, 0, False, True, True, True)
        gen_iter(ebase + np.int32(EC), 1, True, True, True, True)

        def super_body(k2, carry):
            kbase = ebase + k2 * np.int32(2 * EC)
            gen_iter(kbase, 0, True, True, True, True)
            gen_iter(kbase + np.int32(EC), 1, True, True, True, True)
            return carry
        lax.fori_loop(np.int32(1), np.int32(NCH2 - 1), super_body,
                      np.int32(0))

        kbase = ebase + np.int32((NCH2 - 1) * 2 * EC)
        gen_iter(kbase, 0, True, True, False, True)
        gen_iter(kbase + np.int32(EC), 1, True, False, False, False)
        wait_scatter(1)

    def copyout_cur(dstref):
        for j in range(NRC):
            rr = r0 + np.int32(j * RC)
            pltpu.sync_copy(acc.at[pl.ds(rr, RC)], cb1)
            pltpu.sync_copy(cb1, dstref.at[pl.ds(coff + rr, RC)])

    def copyout_final():
        third = np.float32(1.0 / (LAYERS + 1))

        def addin(e, carry):
            for kk in range(H // 16):
                cb1[e, pl.ds(kk * 16, 16)] = (cb1[e, pl.ds(kk * 16, 16)]
                                              + cb2[e, pl.ds(kk * 16, 16)])
            return carry

        def addscale(e, carry):
            for kk in range(H // 16):
                cb1[e, pl.ds(kk * 16, 16)] = (cb1[e, pl.ds(kk * 16, 16)]
                                              + cb2[e, pl.ds(kk * 16, 16)]) * third
            return carry

        for j in range(NRC):
            rr = r0 + np.int32(j * RC)
            pltpu.sync_copy(acc.at[pl.ds(rr, RC)], cb1)
            pltpu.sync_copy(ego.at[pl.ds(coff + rr, RC)], cb2)
            lax.fori_loop(np.int32(0), np.int32(RC), addin, np.int32(0))
            pltpu.sync_copy(cur1.at[pl.ds(coff + rr, RC)], cb2)
            lax.fori_loop(np.int32(0), np.int32(RC), addscale, np.int32(0))
            pltpu.sync_copy(cb1, final.at[pl.ds(coff + rr, RC)])

    zero_cb1()
    zero_acc()
    plsc.subcore_barrier()
    edge_pass(ego)
    plsc.subcore_barrier()
    copyout_cur(cur1)
    zero_cb1()
    zero_acc()
    plsc.subcore_barrier()
    edge_pass(cur1)
    plsc.subcore_barrier()
    copyout_final()


def _propagate_sc(ego, src, dst, w):
    e = src.shape[0]
    ego_p = jnp.pad(ego, ((0, NP - N_NODES), (0, 0)))
    ego_h = ego_p.reshape(NP, 2, H).transpose(1, 0, 2).reshape(2 * NP, H)
    pad = EP - e
    ar = jnp.arange(pad, dtype=jnp.int32)
    src_p = jnp.concatenate([src, ar % N_NODES])
    dst_p = jnp.concatenate([dst, N_NODES + ar % (NP - N_NODES)])
    w_p = jnp.concatenate([w, jnp.zeros((pad,), jnp.float32)])

    mesh = plsc.VectorSubcoreMesh(core_axis_name="c", subcore_axis_name="s")
    out_final, _cur1 = pl.kernel(
        _prop_body,
        out_type=(jax.ShapeDtypeStruct((2 * NP, H), jnp.float32),
                  jax.ShapeDtypeStruct((2 * NP, H), jnp.float32)),
        mesh=mesh,
        scratch_types=[
            pltpu.VMEM_SHARED((NP, H), jnp.float32),
            pltpu.VMEM((EC, H), jnp.float32),
            pltpu.VMEM((EC, H), jnp.float32),
            pltpu.VMEM((EC, H), jnp.float32),
            pltpu.VMEM((EC,), jnp.int32),
            pltpu.VMEM((EC,), jnp.int32),
            pltpu.VMEM((EC,), jnp.float32),
            pltpu.VMEM((EC,), jnp.float32),
            pltpu.VMEM((EC,), jnp.int32),
            pltpu.VMEM((EC,), jnp.int32),
            pltpu.VMEM((RC, H), jnp.float32),
            pltpu.VMEM((RC, H), jnp.float32),
            pltpu.SemaphoreType.DMA,
            pltpu.SemaphoreType.DMA,
            pltpu.SemaphoreType.DMA,
            pltpu.SemaphoreType.DMA,
            pltpu.SemaphoreType.DMA,
            pltpu.SemaphoreType.DMA,
            pltpu.SemaphoreType.DMA,
            pltpu.SemaphoreType.DMA,
        ],
        compiler_params=pltpu.CompilerParams(use_tc_tiling_on_sc=False),
    )(ego_h, src_p, dst_p, w_p)
    emb = out_final.reshape(2, NP, H).transpose(1, 0, 2).reshape(NP, D)
    return emb[:N_NODES]


PB = 4096
PT = PB // (SC_NC * SC_NS)


def _dots_body(ua, ia, ub, ib, iua, iia, iub, iib, pos_a, pos_b,
               iu_v, ii_v, ru_v, ri_v, po_v, sem):
    c = lax.axis_index("c")
    s = lax.axis_index("s")
    wid = s * np.int32(SC_NC) + c
    base = wid * np.int32(PT)

    def one_domain(ut, it, iu, ii, pos):
        pltpu.sync_copy(iu.at[pl.ds(base, PT)], iu_v)
        pltpu.sync_copy(ii.at[pl.ds(base, PT)], ii_v)
        pltpu.async_copy(ut.at[iu_v], ru_v, sem)
        pltpu.async_copy(it.at[ii_v], ri_v, sem)
        pltpu.make_async_copy(ut.at[iu_v], ru_v, sem).wait()
        pltpu.make_async_copy(it.at[ii_v], ri_v, sem).wait()

        lanes = lax.broadcasted_iota(jnp.int32, (16,), 0)

        def pbody(g, carry):
            g0 = g * np.int32(16)
            vec = jnp.zeros((16,), jnp.float32)
            for j in range(16):
                p = g0 + j
                acc = ru_v[p, pl.ds(0, 16)] * ri_v[p, pl.ds(0, 16)]
                for kk in range(1, D // 16):
                    acc = acc + (ru_v[p, pl.ds(kk * 16, 16)]
                                 * ri_v[p, pl.ds(kk * 16, 16)])
                vec = jnp.where(lanes == j, jnp.sum(acc), vec)
            po_v[pl.ds(g0, 16)] = vec
            return carry
        lax.fori_loop(np.int32(0), np.int32(PT // 16), pbody, np.int32(0))
        pltpu.sync_copy(po_v, pos.at[pl.ds(base, PT)])

    one_domain(ua, ia, iua, iia, pos_a)
    one_domain(ub, ib, iub, iib, pos_b)


def _dots_sc(uA, iA, uB, iB, iua, iia, iub, iib):
    mesh = plsc.VectorSubcoreMesh(core_axis_name="c", subcore_axis_name="s")
    pos_a, pos_b = pl.kernel(
        _dots_body,
        out_type=(jax.ShapeDtypeStruct((PB,), jnp.float32),
                  jax.ShapeDtypeStruct((PB,), jnp.float32)),
        mesh=mesh,
        scratch_types=[
            pltpu.VMEM((PT,), jnp.int32),
            pltpu.VMEM((PT,), jnp.int32),
            pltpu.VMEM((PT, D), jnp.float32),
            pltpu.VMEM((PT, D), jnp.float32),
            pltpu.VMEM((PT,), jnp.float32),
            pltpu.SemaphoreType.DMA,
        ],
        compiler_params=pltpu.CompilerParams(needs_layout_passes=False),
    )(uA, iA, uB, iB, iua, iia, iub, iib)
    return pos_a, pos_b


def kernel(uEmb_a, iEmb_a, uEmb_b, iEmb_b, edge_weight_a, edge_weight_b,
           edge_index_a, edge_index_b, data_a, data_b):
    with _jcfg.enable_x64(False):
        return _kernel_32(uEmb_a, iEmb_a, uEmb_b, iEmb_b, edge_weight_a,
                          edge_weight_b, edge_index_a, edge_index_b,
                          data_a, data_b)


def _kernel_32(uEmb_a, iEmb_a, uEmb_b, iEmb_b, edge_weight_a, edge_weight_b,
               edge_index_a, edge_index_b, data_a, data_b):
    ei_a = edge_index_a.astype(jnp.int32)
    ei_b = edge_index_b.astype(jnp.int32)
    da = data_a.astype(jnp.int32)
    db = data_b.astype(jnp.int32)

    ego_a = jnp.concatenate([uEmb_a, iEmb_a], axis=0)
    ego_b = jnp.concatenate([uEmb_b, iEmb_b], axis=0)

    emb_a = _propagate_sc(ego_a, ei_a[0], ei_a[1], edge_weight_a)
    emb_b = _propagate_sc(ego_b, ei_b[0], ei_b[1], edge_weight_b)
    ua, iA = emb_a[:N_USERS], emb_a[N_USERS:]
    ub, iB = emb_b[:N_USERS], emb_b[N_USERS:]

    uB = _similarity_transfer(ub, ua)
    uA = _similarity_transfer(ua, ub)

    pos_a, pos_b = _dots_sc(uA, iA, uB, iB, da[0], da[1], db[0], db[1])
    return (pos_a, pos_b)


# speedup vs baseline: 2.3630x; 1.0455x over previous
import functools

import jax
import jax.numpy as jnp
import numpy as np
from jax import lax
from jax.experimental import pallas as pl
from jax.experimental.pallas import tpu as pltpu
from jax.experimental.pallas import tpu_sc as plsc
from jax._src import config as _jcfg

N_USERS = 15000
N_ITEMS = 10000
N_NODES = N_USERS + N_ITEMS
D = 128
LAYERS = 2

NEG_INF = np.float32(-1e30)
EPS = np.float32(1e-12)


def _flash_body(q_ref, k_ref, b_ref, o_ref, acc_ref, l_ref, qn_ref,
                kn_ref, kb_ref):
    i = pl.program_id(0)
    j = pl.program_id(1)
    nj = pl.num_programs(1)
    bk = k_ref.shape[0]

    @pl.when(j == 0)
    def _init():
        acc_ref[...] = jnp.zeros_like(acc_ref)
        l_ref[...] = jnp.zeros_like(l_ref)
        q = q_ref[...]
        qn = q * lax.reciprocal(
            jnp.sqrt(jnp.sum(q * q, axis=1, keepdims=True)) + EPS)
        qn_ref[...] = qn.astype(jnp.bfloat16)

    @pl.when(i == 0)
    def _knorm():
        k = k_ref[...]
        kn = k * lax.reciprocal(
            jnp.sqrt(jnp.sum(k * k, axis=1, keepdims=True)) + EPS)
        kn_ref[pl.ds(j * bk, bk), :] = kn.astype(jnp.bfloat16)
        kb_ref[pl.ds(j * bk, bk), :] = k.astype(jnp.bfloat16)

    s = jax.lax.dot_general(qn_ref[...], kn_ref[pl.ds(j * bk, bk), :],
                            (((1,), (1,)), ((), ())),
                            preferred_element_type=jnp.float32)
    p = jnp.exp(s + b_ref[...])
    l_ref[...] = l_ref[...] + jnp.sum(p, axis=1, keepdims=True)
    pv = jax.lax.dot_general(p.astype(jnp.bfloat16),
                             kb_ref[pl.ds(j * bk, bk), :],
                             (((1,), (0,)), ((), ())),
                             preferred_element_type=jnp.float32)
    acc_ref[...] = acc_ref[...] + pv

    @pl.when(j == nj - 1)
    def _fin():
        o_ref[...] = q_ref[...] + acc_ref[...] * lax.reciprocal(l_ref[...])


def _similarity_transfer(target, auxi, bq=512, bk=1024):
    n, d = target.shape
    nk = auxi.shape[0]
    n_pad = (n + bq - 1) // bq * bq
    nk_pad = (nk + bk - 1) // bk * bk
    tq = jnp.pad(target, ((0, n_pad - n), (0, 0)))
    ta = jnp.pad(auxi, ((0, nk_pad - nk), (0, 0)))
    bias = jnp.asarray(
        np.where(np.arange(nk_pad) < nk, np.float32(-1.0),
                 np.float32(-1e30)).astype(np.float32).reshape(1, nk_pad))
    grid = (n_pad // bq, nk_pad // bk)
    out = pl.pallas_call(
        _flash_body,
        grid=grid,
        in_specs=[
            pl.BlockSpec((bq, d), lambda i, j: (i, np.int32(0))),
            pl.BlockSpec((bk, d), lambda i, j: (j, np.int32(0))),
            pl.BlockSpec((1, bk), lambda i, j: (np.int32(0), j)),
        ],
        out_specs=pl.BlockSpec((bq, d), lambda i, j: (i, np.int32(0))),
        out_shape=jax.ShapeDtypeStruct((n_pad, d), jnp.float32),
        scratch_shapes=[
            pltpu.VMEM((bq, d), jnp.float32),
            pltpu.VMEM((bq, 1), jnp.float32),
            pltpu.VMEM((bq, d), jnp.bfloat16),
            pltpu.VMEM((nk_pad, d), jnp.bfloat16),
            pltpu.VMEM((nk_pad, d), jnp.bfloat16),
        ],
        compiler_params=pltpu.CompilerParams(
            dimension_semantics=("arbitrary", "arbitrary"),
        ),
    )(tq, ta, bias)
    return out[:n]


SC_NC = 2
SC_NS = 16
NP = 25088
H = D // 2
EP = 409600
ET = EP // SC_NS
EC = 128
NCH = ET // EC
RPT = NP // SC_NS
RC = 28
NRC = RPT // RC


def _prop_body(ego, srcr, dstr, wr, final, cur1,
               acc, rows0, rows1, msg_v, src0, src1,
               w0, w1, dst0, dst1, cb1, cb2,
               isem0, isem1, dsem0, dsem1,
               gsem0, gsem1, ssem0, ssem1, wsem0, wsem1, rsem):
    c = lax.axis_index("c")
    s = lax.axis_index("s")
    r0 = s * np.int32(RPT)
    ebase = s * np.int32(ET)
    coff = c * np.int32(NP)

    def zero_cb1():
        def zb(e, carry):
            for kk in range(H // 16):
                cb1[e, pl.ds(kk * 16, 16)] = jnp.zeros((16,), jnp.float32)
            return carry
        lax.fori_loop(np.int32(0), np.int32(RC), zb, np.int32(0))

    def zero_acc():
        for j in range(NRC):
            pltpu.sync_copy(cb1, acc.at[pl.ds(r0 + np.int32(j * RC), RC)])

    def edge_pass(tab):
        srcs = (src0, src1)
        wbufs = (w0, w1)
        dsts = (dst0, dst1)
        rows = (rows0, rows1)
        isems = (isem0, isem1)
        dsems = (dsem0, dsem1)
        gsems = (gsem0, gsem1)
        ssems = (ssem0, ssem1)
        NCH2 = NCH // 2

        def fire_srcw(koff, j):
            pltpu.async_copy(srcr.at[pl.ds(koff, EC)], srcs[j], isems[j])
            pltpu.async_copy(wr.at[pl.ds(koff, EC)], wbufs[j], isems[j])

        def wait_srcw(j):
            pltpu.make_async_copy(srcr.at[pl.ds(ebase, EC)], srcs[j],
                                  isems[j]).wait()
            pltpu.make_async_copy(wr.at[pl.ds(ebase, EC)], wbufs[j],
                                  isems[j]).wait()

        def fire_dst(koff, b):
            pltpu.async_copy(dstr.at[pl.ds(koff, EC)], dsts[b], dsems[b])

        def wait_dst(b):
            pltpu.make_async_copy(dstr.at[pl.ds(ebase, EC)], dsts[b],
                                  dsems[b]).wait()

        def adjust(j):
            for g in range(EC // 16):
                srcs[j][pl.ds(g * 16, 16)] = srcs[j][pl.ds(g * 16, 16)] + coff

        def fire_gather(j, b):
            pltpu.async_copy(tab.at[srcs[j]], rows[b], gsems[b])

        def wait_gather(j, b):
            pltpu.make_async_copy(tab.at[srcs[j]], rows[b], gsems[b]).wait()

        def fire_scatter(b):
            pltpu.async_copy(msg_v, acc.at[dsts[b]], ssems[b], add=True)

        def wait_scatter(b):
            pltpu.make_async_copy(msg_v, acc.at[dsts[b]], ssems[b]).wait()

        def weight(j, b):
            def wbody(g, carry):
                e0g = g * np.int32(16)
                wv = wbufs[j][pl.ds(e0g, 16)]
                for jj in range(16):
                    wsc = wv[jj]
                    for kk in range(H // 16):
                        msg_v[e0g + jj, pl.ds(kk * 16, 16)] = (
                            rows[b][e0g + jj, pl.ds(kk * 16, 16)] * wsc)
                return carry
            lax.fori_loop(np.int32(0), np.int32(EC // 16), wbody, np.int32(0))

        def gen_iter(koff, b, wait_prev, fire_dst_nxt, fire_srcw2,
                     next_gather):
            if wait_prev:
                wait_scatter(b ^ 1)
            if fire_dst_nxt:
                fire_dst(koff + np.int32(EC), b ^ 1)
            wait_gather(b, b)
            wait_dst(b)
            weight(b, b)
            if fire_srcw2:
                fire_srcw(koff + np.int32(2 * EC), b)
            fire_scatter(b)
            if next_gather:
                wait_srcw(b ^ 1)
                adjust(b ^ 1)
                fire_gather(b ^ 1, b ^ 1)

        fire_srcw(ebase, 0)
        fire_srcw(ebase + np.int32(EC), 1)
        fire_dst(ebase, 0)
        wait_srcw(0)
        adjust(0)
        fire_gather(0, 0)
        gen_iter(ebase, 0, False, True, True, True)
        gen_iter(ebase + np.int32(EC), 1, True, True, True, True)

        def super_body(k2, carry):
            kbase = ebase + k2 * np.int32(2 * EC)
            gen_iter(kbase, 0, True, True, True, True)
            gen_iter(kbase + np.int32(EC), 1, True, True, True, True)
            return carry
        lax.fori_loop(np.int32(1), np.int32(NCH2 - 1), super_body,
                      np.int32(0))

        kbase = ebase + np.int32((NCH2 - 1) * 2 * EC)
        gen_iter(kbase, 0, True, True, False, True)
        gen_iter(kbase + np.int32(EC), 1, True, False, False, False)
        wait_scatter(1)

    def copyout_cur(dstref):
        wsems = (wsem0, wsem1)
        cbs = (cb1, cb2)
        for j in range(NRC):
            cb = cbs[j % 2]
            rr = r0 + np.int32(j * RC)
            pltpu.sync_copy(acc.at[pl.ds(rr, RC)], cb)
            pltpu.sync_copy(cb, dstref.at[pl.ds(coff + rr, RC)])

    def copyout_final():
        third = np.float32(1.0 / (LAYERS + 1))

        def addscale(e, carry):
            for kk in range(H // 16):
                cb1[e, pl.ds(kk * 16, 16)] = (
                    cb1[e, pl.ds(kk * 16, 16)]
                    + cb2[e, pl.ds(kk * 16, 16)]) * third
            return carry

        for j in range(NRC):
            rr = r0 + np.int32(j * RC)
            pltpu.sync_copy(acc.at[pl.ds(rr, RC)], cb1)
            pltpu.sync_copy(ego.at[pl.ds(coff + rr, RC)], cb2)
            lax.fori_loop(np.int32(0), np.int32(RC), addscale, np.int32(0))
            pltpu.sync_copy(cb1, final.at[pl.ds(coff + rr, RC)])

    zero_cb1()
    zero_acc()
    plsc.subcore_barrier()
    edge_pass(ego)
    plsc.subcore_barrier()
    copyout_cur(cur1)
    plsc.subcore_barrier()
    edge_pass(cur1)
    plsc.subcore_barrier()
    copyout_final()


def _propagate_sc(ego, src, dst, w):
    e = src.shape[0]
    ego_p = jnp.pad(ego, ((0, NP - N_NODES), (0, 0)))
    ego_h = ego_p.reshape(NP, 2, H).transpose(1, 0, 2).reshape(2 * NP, H)
    pad = EP - e
    ar = jnp.arange(pad, dtype=jnp.int32)
    src_p = jnp.concatenate([src, ar % N_NODES])
    dst_p = jnp.concatenate([dst, N_NODES + ar % (NP - N_NODES)])
    w_p = jnp.concatenate([w, jnp.zeros((pad,), jnp.float32)])

    mesh = plsc.VectorSubcoreMesh(core_axis_name="c", subcore_axis_name="s")
    out_final, _cur1 = pl.kernel(
        _prop_body,
        out_type=(jax.ShapeDtypeStruct((2 * NP, H), jnp.float32),
                  jax.ShapeDtypeStruct((2 * NP, H), jnp.float32)),
        mesh=mesh,
        scratch_types=[
            pltpu.VMEM_SHARED((NP, H), jnp.float32),
            pltpu.VMEM((EC, H), jnp.float32),
            pltpu.VMEM((EC, H), jnp.float32),
            pltpu.VMEM((EC, H), jnp.float32),
            pltpu.VMEM((EC,), jnp.int32),
            pltpu.VMEM((EC,), jnp.int32),
            pltpu.VMEM((EC,), jnp.float32),
            pltpu.VMEM((EC,), jnp.float32),
            pltpu.VMEM((EC,), jnp.int32),
            pltpu.VMEM((EC,), jnp.int32),
            pltpu.VMEM((RC, H), jnp.float32),
            pltpu.VMEM((RC, H), jnp.float32),
            pltpu.SemaphoreType.DMA,
            pltpu.SemaphoreType.DMA,
            pltpu.SemaphoreType.DMA,
            pltpu.SemaphoreType.DMA,
            pltpu.SemaphoreType.DMA,
            pltpu.SemaphoreType.DMA,
            pltpu.SemaphoreType.DMA,
            pltpu.SemaphoreType.DMA,
            pltpu.SemaphoreType.DMA,
            pltpu.SemaphoreType.DMA,
            pltpu.SemaphoreType.DMA,
        ],
        compiler_params=pltpu.CompilerParams(use_tc_tiling_on_sc=False),
    )(ego_h, src_p, dst_p, w_p)
    emb = out_final.reshape(2, NP, H).transpose(1, 0, 2).reshape(NP, D)
    return emb[:N_NODES]


PB = 4096
PT = PB // (SC_NC * SC_NS)


def _dots_body(ua, ia, ub, ib, iua, iia, iub, iib, pos_a, pos_b,
               iu_v, ii_v, ru_v, ri_v, po_v, sem):
    c = lax.axis_index("c")
    s = lax.axis_index("s")
    wid = s * np.int32(SC_NC) + c
    base = wid * np.int32(PT)

    def one_domain(ut, it, iu, ii, pos):
        pltpu.sync_copy(iu.at[pl.ds(base, PT)], iu_v)
        pltpu.sync_copy(ii.at[pl.ds(base, PT)], ii_v)
        pltpu.async_copy(ut.at[iu_v], ru_v, sem)
        pltpu.async_copy(it.at[ii_v], ri_v, sem)
        pltpu.make_async_copy(ut.at[iu_v], ru_v, sem).wait()
        pltpu.make_async_copy(it.at[ii_v], ri_v, sem).wait()

        lanes = lax.broadcasted_iota(jnp.int32, (16,), 0)

        def pbody(g, carry):
            g0 = g * np.int32(16)
            vec = jnp.zeros((16,), jnp.float32)
            for j in range(16):
                p = g0 + j
                acc = ru_v[p, pl.ds(0, 16)] * ri_v[p, pl.ds(0, 16)]
                for kk in range(1, D // 16):
                    acc = acc + (ru_v[p, pl.ds(kk * 16, 16)]
                                 * ri_v[p, pl.ds(kk * 16, 16)])
                vec = jnp.where(lanes == j, jnp.sum(acc), vec)
            po_v[pl.ds(g0, 16)] = vec
            return carry
        lax.fori_loop(np.int32(0), np.int32(PT // 16), pbody, np.int32(0))
        pltpu.sync_copy(po_v, pos.at[pl.ds(base, PT)])

    one_domain(ua, ia, iua, iia, pos_a)
    one_domain(ub, ib, iub, iib, pos_b)


def _dots_sc(uA, iA, uB, iB, iua, iia, iub, iib):
    mesh = plsc.VectorSubcoreMesh(core_axis_name="c", subcore_axis_name="s")
    pos_a, pos_b = pl.kernel(
        _dots_body,
        out_type=(jax.ShapeDtypeStruct((PB,), jnp.float32),
                  jax.ShapeDtypeStruct((PB,), jnp.float32)),
        mesh=mesh,
        scratch_types=[
            pltpu.VMEM((PT,), jnp.int32),
            pltpu.VMEM((PT,), jnp.int32),
            pltpu.VMEM((PT, D), jnp.float32),
            pltpu.VMEM((PT, D), jnp.float32),
            pltpu.VMEM((PT,), jnp.float32),
            pltpu.SemaphoreType.DMA,
        ],
        compiler_params=pltpu.CompilerParams(needs_layout_passes=False),
    )(uA, iA, uB, iB, iua, iia, iub, iib)
    return pos_a, pos_b


def kernel(uEmb_a, iEmb_a, uEmb_b, iEmb_b, edge_weight_a, edge_weight_b,
           edge_index_a, edge_index_b, data_a, data_b):
    with _jcfg.enable_x64(False):
        return _kernel_32(uEmb_a, iEmb_a, uEmb_b, iEmb_b, edge_weight_a,
                          edge_weight_b, edge_index_a, edge_index_b,
                          data_a, data_b)


def _kernel_32(uEmb_a, iEmb_a, uEmb_b, iEmb_b, edge_weight_a, edge_weight_b,
               edge_index_a, edge_index_b, data_a, data_b):
    ei_a = edge_index_a.astype(jnp.int32)
    ei_b = edge_index_b.astype(jnp.int32)
    da = data_a.astype(jnp.int32)
    db = data_b.astype(jnp.int32)

    ego_a = jnp.concatenate([uEmb_a, iEmb_a], axis=0)
    ego_b = jnp.concatenate([uEmb_b, iEmb_b], axis=0)

    emb_a = _propagate_sc(ego_a, ei_a[0], ei_a[1], edge_weight_a)
    emb_b = _propagate_sc(ego_b, ei_b[0], ei_b[1], edge_weight_b)
    ua, iA = emb_a[:N_USERS], emb_a[N_USERS:]
    ub, iB = emb_b[:N_USERS], emb_b[N_USERS:]

    uB = _similarity_transfer(ub, ua)
    uA = _similarity_transfer(ua, ub)

    pos_a, pos_b = _dots_sc(uA, iA, uB, iB, da[0], da[1], db[0], db[1])
    return (pos_a, pos_b)
